# baseline (speedup 1.0000x reference)
import functools

import jax
import jax.numpy as jnp
from jax import lax
from jax.experimental import pallas as pl
from jax.experimental.pallas import tpu as pltpu
from jax.experimental.pallas import tpu_sc as plsc

N_USERS = 50000
N_NODES = 100000
NP = 100096
E = 1600000
EP = 1646592
H = 16
NC, NS = 2, 16
EPT = EP // NS
C = 512
NCHUNK = EPT // C
SUB = 128
NSUB = C // SUB
B = 16384
BPT = B // NS
RPT = NP // NS
ROWB = EPT // SUB
NT = NCHUNK // 3


def _gcn_body(e0, ei3, users4, items4,
              e1, e2, uemb, iemb,
              acc, rowsA, rowsB, rowsC, idxA, idxB, idxC,
              sidxA, sidxB, sidxC,
              isemA, isemB, isemC, gsemA, gsemB, gsemC,
              ssemA, ssemB, ssemC):
    cid = lax.axis_index("c")
    sid = lax.axis_index("s")
    base_node = cid * NP
    ROWS = (rowsA, rowsB, rowsC)
    IDXB = (idxA, idxB, idxC)
    SIDX = (sidxA, sidxB, sidxC)
    ISEM = (isemA, isemB, isemC)
    GSEM = (gsemA, gsemB, gsemC)
    SSEM = (ssemA, ssemB, ssemC)

    def propagate(e_in, e_out):
        @plsc.parallel_loop(0, C, unroll=4)
        def _(i):
            z = jnp.zeros((H,), jnp.float32)
            rowsA[i] = z
            rowsB[i] = z
            rowsC[i] = z

        for t in range(12):
            pltpu.sync_copy(ROWS[t % 3],
                            acc.at[pl.ds(sid * RPT + t * C, C)])
        pltpu.sync_copy(rowsA.at[pl.ds(0, RPT - 12 * C)],
                        acc.at[pl.ds(sid * RPT + 12 * C, RPT - 12 * C)])
        plsc.subcore_barrier()

        def fire_idx(c, b):
            pltpu.async_copy(
                ei3.at[pl.ds(sid * ROWB + c * NSUB, NSUB)], IDXB[b], ISEM[b])

        def wait_idx(b):
            pltpu.make_async_copy(
                ei3.at[pl.ds(0, NSUB)], IDXB[b], ISEM[b]).wait()

        def rebase(b):
            @plsc.parallel_loop(0, NSUB)
            def _(j):
                for g in range(SUB // 16):
                    IDXB[b][j, 0, pl.ds(g * 16, 16)] = (
                        IDXB[b][j, 0, pl.ds(g * 16, 16)] + base_node)

        def fire_gathers(b):
            for j in range(NSUB):
                pltpu.async_copy(e_in.at[IDXB[b].at[j, 0]],
                                 ROWS[b].at[pl.ds(j * SUB, SUB)], GSEM[b])

        def weight_and_scatter(b):
            for j in range(NSUB):
                pltpu.make_async_copy(
                    e_in.at[IDXB[b].at[j, 0]],
                    ROWS[b].at[pl.ds(j * SUB, SUB)], GSEM[b]).wait()

                @plsc.parallel_loop(0, SUB // 16)
                def _(gg, j=j):
                    SIDX[b][j, pl.ds(gg * 16, 16)] = (
                        IDXB[b][j, 1, pl.ds(gg * 16, 16)])
                    wv = plsc.bitcast(IDXB[b][j, 2, pl.ds(gg * 16, 16)],
                                      jnp.float32)
                    base = j * SUB + gg * 16
                    for e in range(16):
                        ROWS[b][base + e] = ROWS[b][base + e] * wv[e]

                pltpu.async_copy(ROWS[b].at[pl.ds(j * SUB, SUB)],
                                 acc.at[SIDX[b].at[j]], SSEM[b], add=True)

        def drain_scatters(b):
            pltpu.make_async_copy(
                ROWS[b], acc.at[pl.ds(0, C)], SSEM[b]).wait()

        fire_idx(0, 0)
        fire_idx(1, 1)

        @pl.loop(0, NT)
        def _(t):
            for b in range(3):
                wait_idx(b)

                if b == 2:
                    drain_scatters(0)
                else:
                    @pl.when(t >= 1)
                    def _(b=b):
                        drain_scatters((b + 1) % 3)

                rebase(b)
                fire_gathers(b)

                pb = (b + 2) % 3
                if b == 0:
                    @pl.when(t >= 1)
                    def _():
                        weight_and_scatter(2)
                else:
                    weight_and_scatter(pb)

                if b == 0:
                    fire_idx(3 * t + 2, 2)
                else:
                    @pl.when(t + 1 < NT)
                    def _(b=b):
                        fire_idx(3 * t + b + 2, (b + 2) % 3)

        weight_and_scatter(2)
        drain_scatters(1)
        drain_scatters(2)

        plsc.subcore_barrier()
        pltpu.sync_copy(acc.at[pl.ds(sid * RPT, RPT)],
                        e_out.at[pl.ds(base_node + sid * RPT, RPT)])
        plsc.subcore_barrier()

    propagate(e0, e1)
    propagate(e1, e2)

    third = jnp.float32(1.0 / 3.0)

    def lookup(src4, offset, emb_out):
        pltpu.sync_copy(src4.at[sid, 0], sidxA)
        pltpu.sync_copy(src4.at[sid, 1], sidxB)

        for sb in (sidxA, sidxB):
            @plsc.parallel_loop(0, NSUB)
            def _(j, sb=sb):
                for g in range(SUB // 16):
                    sb[j, pl.ds(g * 16, 16)] = (
                        sb[j, pl.ds(g * 16, 16)] + (base_node + offset))

        for half, sb in enumerate((sidxA, sidxB)):
            @pl.loop(0, NSUB)
            def _(f, half=half, sb=sb):
                for ti, tbl in enumerate((e0, e1, e2)):
                    pltpu.async_copy(tbl.at[sb.at[f]],
                                     rowsA.at[pl.ds(ti * SUB, SUB)], gsemA)
                for ti, tbl in enumerate((e0, e1, e2)):
                    pltpu.make_async_copy(
                        tbl.at[sb.at[f]],
                        rowsA.at[pl.ds(ti * SUB, SUB)], gsemA).wait()

                @plsc.parallel_loop(0, SUB, unroll=4)
                def _(i):
                    rowsA[3 * SUB + i] = (
                        (rowsA[i] + rowsA[SUB + i] + rowsA[2 * SUB + i])
                        * third)

                pltpu.sync_copy(
                    rowsA.at[pl.ds(3 * SUB, SUB)],
                    emb_out.at[pl.ds(
                        cid * B + sid * BPT + (half * NSUB + f) * SUB, SUB)])

    lookup(users4, 0, uemb)
    lookup(items4, N_USERS, iemb)


_MESH = plsc.VectorSubcoreMesh(core_axis_name="c", subcore_axis_name="s",
                               num_cores=NC, num_subcores=NS)

_gcn = functools.partial(
    pl.kernel,
    out_type=(
        jax.ShapeDtypeStruct((2 * NP, H), jnp.float32),
        jax.ShapeDtypeStruct((2 * NP, H), jnp.float32),
        jax.ShapeDtypeStruct((2 * B, H), jnp.float32),
        jax.ShapeDtypeStruct((2 * B, H), jnp.float32),
    ),
    mesh=_MESH,
    scratch_types=[
        pltpu.VMEM_SHARED((NP, H), jnp.float32),
        pltpu.VMEM((C, H), jnp.float32),
        pltpu.VMEM((C, H), jnp.float32),
        pltpu.VMEM((C, H), jnp.float32),
        pltpu.VMEM((NSUB, 3, SUB), jnp.int32),
        pltpu.VMEM((NSUB, 3, SUB), jnp.int32),
        pltpu.VMEM((NSUB, 3, SUB), jnp.int32),
        pltpu.VMEM((NSUB, SUB), jnp.int32),
        pltpu.VMEM((NSUB, SUB), jnp.int32),
        pltpu.VMEM((NSUB, SUB), jnp.int32),
        pltpu.SemaphoreType.DMA,
        pltpu.SemaphoreType.DMA,
        pltpu.SemaphoreType.DMA,
        pltpu.SemaphoreType.DMA,
        pltpu.SemaphoreType.DMA,
        pltpu.SemaphoreType.DMA,
        pltpu.SemaphoreType.DMA,
        pltpu.SemaphoreType.DMA,
        pltpu.SemaphoreType.DMA,
    ],
    compiler_params=pltpu.CompilerParams(use_tc_tiling_on_sc=False,
                                         needs_layout_passes=False),
)(_gcn_body)


def _scores_body(u_ref, i_ref, o_ref):
    o_ref[...] = jnp.sum(u_ref[...] * i_ref[...], axis=1)


_scores = pl.pallas_call(
    _scores_body,
    out_shape=jax.ShapeDtypeStruct((B,), jnp.float32),
    grid=(8,),
    in_specs=[pl.BlockSpec((B // 8, 32), lambda i: (i, 0))] * 2,
    out_specs=pl.BlockSpec((B // 8,), lambda i: (i,)),
)


def kernel(user_table, item_table, edge_weight, edge_index, users, items):
    all_emb = jnp.concatenate([user_table, item_table], axis=0)
    npad = jnp.zeros((NP - N_NODES, H), jnp.float32)
    e0 = jnp.concatenate(
        [all_emb[:, :H], npad, all_emb[:, H:], npad], axis=0)
    epad = jnp.zeros((EP - E,), jnp.int32)
    ci = edge_index.astype(jnp.int32)
    colp = jnp.concatenate([ci[1], epad]).reshape(-1, 1, SUB)
    dstp = jnp.concatenate([ci[0], epad]).reshape(-1, 1, SUB)
    wbits = jax.lax.bitcast_convert_type(
        jnp.concatenate([edge_weight, jnp.zeros((EP - E,), jnp.float32)]),
        jnp.int32).reshape(-1, 1, SUB)
    ei3 = jnp.concatenate([colp, dstp, wbits], axis=1)
    users4 = users.astype(jnp.int32).reshape(NS, 2, NSUB, SUB)
    items4 = items.astype(jnp.int32).reshape(NS, 2, NSUB, SUB)
    _, _, ue, ie = _gcn(e0, ei3, users4, items4)
    users_emb = jnp.concatenate([ue[:B], ue[B:]], axis=1)
    items_emb = jnp.concatenate([ie[:B], ie[B:]], axis=1)
    scores = _scores(users_emb, items_emb)
    return (users_emb, items_emb, scores)


# speedup vs baseline: 1.0385x; 1.0385x over previous
import functools

import jax
import jax.numpy as jnp
from jax import lax
from jax.experimental import pallas as pl
from jax.experimental.pallas import tpu as pltpu
from jax.experimental.pallas import tpu_sc as plsc

N_USERS = 50000
N_NODES = 100000
NP = 100096
E = 1600000
EP = 1638400
H = 16
NC, NS = 2, 16
EPT = EP // NS
C = 512
NCHUNK = EPT // C
SUB = 128
NSUB = C // SUB
B = 16384
BPT = B // NS
RPT = NP // NS
ROWB = EPT // SUB


def _gcn_body(e0, ei3, users4, items4,
              e1, e2, uemb, iemb,
              acc, rowsA, rowsB, idxA, idxB, sidxA, sidxB,
              isemA, isemB, gsemA, gsemB, ssemA, ssemB):
    cid = lax.axis_index("c")
    sid = lax.axis_index("s")
    base_node = cid * NP
    ROWS = (rowsA, rowsB)
    IDXB = (idxA, idxB)
    SIDX = (sidxA, sidxB)
    ISEM = (isemA, isemB)
    GSEM = (gsemA, gsemB)
    SSEM = (ssemA, ssemB)

    def propagate(e_in, e_out):
        @plsc.parallel_loop(0, C, unroll=4)
        def _(i):
            z = jnp.zeros((H,), jnp.float32)
            rowsA[i] = z
            rowsB[i] = z

        for t in range(12):
            pltpu.sync_copy(ROWS[t % 2],
                            acc.at[pl.ds(sid * RPT + t * C, C)])
        pltpu.sync_copy(rowsA.at[pl.ds(0, RPT - 12 * C)],
                        acc.at[pl.ds(sid * RPT + 12 * C, RPT - 12 * C)])
        plsc.subcore_barrier()

        def fire_idx(c, b):
            pltpu.async_copy(
                ei3.at[pl.ds(sid * ROWB + c * NSUB, NSUB)], IDXB[b], ISEM[b])

        def wait_idx(b):
            pltpu.make_async_copy(
                ei3.at[pl.ds(0, NSUB)], IDXB[b], ISEM[b]).wait()

        def rebase(b):
            @plsc.parallel_loop(0, NSUB)
            def _(j):
                for g in range(SUB // 16):
                    IDXB[b][j, 0, pl.ds(g * 16, 16)] = (
                        IDXB[b][j, 0, pl.ds(g * 16, 16)] + base_node)

        def fire_gathers(b):
            for j in range(NSUB):
                pltpu.async_copy(e_in.at[IDXB[b].at[j, 0]],
                                 ROWS[b].at[pl.ds(j * SUB, SUB)], GSEM[b])

        def weight_and_scatter(b):
            for j in range(NSUB):
                pltpu.make_async_copy(
                    e_in.at[IDXB[b].at[j, 0]],
                    ROWS[b].at[pl.ds(j * SUB, SUB)], GSEM[b]).wait()

                @plsc.parallel_loop(0, SUB // 16)
                def _(gg, j=j):
                    SIDX[b][j, pl.ds(gg * 16, 16)] = (
                        IDXB[b][j, 1, pl.ds(gg * 16, 16)])
                    wv = plsc.bitcast(IDXB[b][j, 2, pl.ds(gg * 16, 16)],
                                      jnp.float32)
                    base = j * SUB + gg * 16
                    for e in range(16):
                        ROWS[b][base + e] = ROWS[b][base + e] * wv[e]

                pltpu.async_copy(ROWS[b].at[pl.ds(j * SUB, SUB)],
                                 acc.at[SIDX[b].at[j]], SSEM[b], add=True)

        def drain_scatters(b):
            pltpu.make_async_copy(
                ROWS[b], acc.at[pl.ds(0, C)], SSEM[b]).wait()

        fire_idx(0, 0)

        @pl.loop(0, NCHUNK // 2)
        def _(t):
            wait_idx(0)

            @pl.when(t >= 1)
            def _():
                drain_scatters(0)

            rebase(0)
            fire_gathers(0)

            @pl.when(t >= 1)
            def _():
                weight_and_scatter(1)

            fire_idx(2 * t + 1, 1)

            wait_idx(1)

            @pl.when(t >= 1)
            def _():
                drain_scatters(1)

            rebase(1)
            fire_gathers(1)

            weight_and_scatter(0)

            @pl.when(t + 1 < NCHUNK // 2)
            def _():
                fire_idx(2 * t + 2, 0)

        weight_and_scatter(1)
        drain_scatters(0)
        drain_scatters(1)

        plsc.subcore_barrier()
        pltpu.sync_copy(acc.at[pl.ds(sid * RPT, RPT)],
                        e_out.at[pl.ds(base_node + sid * RPT, RPT)])
        plsc.subcore_barrier()

    propagate(e0, e1)
    propagate(e1, e2)

    third = jnp.float32(1.0 / 3.0)

    def lookup(src4, offset, emb_out):
        pltpu.sync_copy(src4.at[sid, 0], sidxA)
        pltpu.sync_copy(src4.at[sid, 1], sidxB)

        for sb in (sidxA, sidxB):
            @plsc.parallel_loop(0, NSUB)
            def _(j, sb=sb):
                for g in range(SUB // 16):
                    sb[j, pl.ds(g * 16, 16)] = (
                        sb[j, pl.ds(g * 16, 16)] + (base_node + offset))

        for half, sb in enumerate((sidxA, sidxB)):
            @pl.loop(0, NSUB)
            def _(f, half=half, sb=sb):
                for ti, tbl in enumerate((e0, e1, e2)):
                    pltpu.async_copy(tbl.at[sb.at[f]],
                                     rowsA.at[pl.ds(ti * SUB, SUB)], gsemA)
                for ti, tbl in enumerate((e0, e1, e2)):
                    pltpu.make_async_copy(
                        tbl.at[sb.at[f]],
                        rowsA.at[pl.ds(ti * SUB, SUB)], gsemA).wait()

                @plsc.parallel_loop(0, SUB, unroll=4)
                def _(i):
                    rowsA[3 * SUB + i] = (
                        (rowsA[i] + rowsA[SUB + i] + rowsA[2 * SUB + i])
                        * third)

                pltpu.sync_copy(
                    rowsA.at[pl.ds(3 * SUB, SUB)],
                    emb_out.at[pl.ds(
                        cid * B + sid * BPT + (half * NSUB + f) * SUB, SUB)])

    lookup(users4, 0, uemb)
    lookup(items4, N_USERS, iemb)


_MESH = plsc.VectorSubcoreMesh(core_axis_name="c", subcore_axis_name="s",
                               num_cores=NC, num_subcores=NS)

_gcn = functools.partial(
    pl.kernel,
    out_type=(
        jax.ShapeDtypeStruct((2 * NP, H), jnp.float32),
        jax.ShapeDtypeStruct((2 * NP, H), jnp.float32),
        jax.ShapeDtypeStruct((2 * B, H), jnp.float32),
        jax.ShapeDtypeStruct((2 * B, H), jnp.float32),
    ),
    mesh=_MESH,
    scratch_types=[
        pltpu.VMEM_SHARED((NP, H), jnp.float32),
        pltpu.VMEM((C, H), jnp.float32),
        pltpu.VMEM((C, H), jnp.float32),
        pltpu.VMEM((NSUB, 3, SUB), jnp.int32),
        pltpu.VMEM((NSUB, 3, SUB), jnp.int32),
        pltpu.VMEM((NSUB, SUB), jnp.int32),
        pltpu.VMEM((NSUB, SUB), jnp.int32),
        pltpu.SemaphoreType.DMA,
        pltpu.SemaphoreType.DMA,
        pltpu.SemaphoreType.DMA,
        pltpu.SemaphoreType.DMA,
        pltpu.SemaphoreType.DMA,
        pltpu.SemaphoreType.DMA,
    ],
    compiler_params=pltpu.CompilerParams(use_tc_tiling_on_sc=False,
                                         needs_layout_passes=False),
)(_gcn_body)


def _scores_body(u_ref, i_ref, o_ref):
    o_ref[...] = jnp.sum(u_ref[...] * i_ref[...], axis=1)


_scores = pl.pallas_call(
    _scores_body,
    out_shape=jax.ShapeDtypeStruct((B,), jnp.float32),
    grid=(8,),
    in_specs=[pl.BlockSpec((B // 8, 32), lambda i: (i, 0))] * 2,
    out_specs=pl.BlockSpec((B // 8,), lambda i: (i,)),
)


def kernel(user_table, item_table, edge_weight, edge_index, users, items):
    all_emb = jnp.concatenate([user_table, item_table], axis=0)
    npad = jnp.zeros((NP - N_NODES, H), jnp.float32)
    e0 = jnp.concatenate(
        [all_emb[:, :H], npad, all_emb[:, H:], npad], axis=0)
    epad = jnp.zeros((EP - E,), jnp.int32)
    ci = edge_index.astype(jnp.int32)
    colp = jnp.concatenate([ci[1], epad]).reshape(-1, 1, SUB)
    dstp = jnp.concatenate([ci[0], epad]).reshape(-1, 1, SUB)
    wbits = jax.lax.bitcast_convert_type(
        jnp.concatenate([edge_weight, jnp.zeros((EP - E,), jnp.float32)]),
        jnp.int32).reshape(-1, 1, SUB)
    ei3 = jnp.concatenate([colp, dstp, wbits], axis=1)
    users4 = users.astype(jnp.int32).reshape(NS, 2, NSUB, SUB)
    items4 = items.astype(jnp.int32).reshape(NS, 2, NSUB, SUB)
    _, _, ue, ie = _gcn(e0, ei3, users4, items4)
    users_emb = jnp.concatenate([ue[:B], ue[B:]], axis=1)
    items_emb = jnp.concatenate([ie[:B], ie[B:]], axis=1)
    scores = _scores(users_emb, items_emb)
    return (users_emb, items_emb, scores)


# speedup vs baseline: 1.0534x; 1.0143x over previous
import functools

import jax
import jax.numpy as jnp
from jax import lax
from jax.experimental import pallas as pl
from jax.experimental.pallas import tpu as pltpu
from jax.experimental.pallas import tpu_sc as plsc

N_USERS = 50000
N_NODES = 100000
NP = 100096
E = 1600000
EP = 1638400
H = 16
NC, NS = 2, 16
EPT = EP // NS
C = 640
NCHUNK = EPT // C
SUB = 128
NSUB = C // SUB
FN = 4
B = 16384
BPT = B // NS
RPT = NP // NS
ROWB = EPT // SUB


def _gcn_body(e0, ei3, users4, items4,
              e1, e2, uemb, iemb,
              acc, rowsA, rowsB, idxA, idxB, sidxA, sidxB,
              isemA, isemB, gsemA, gsemB, ssemA, ssemB):
    cid = lax.axis_index("c")
    sid = lax.axis_index("s")
    base_node = cid * NP
    ROWS = (rowsA, rowsB)
    IDXB = (idxA, idxB)
    SIDX = (sidxA, sidxB)
    ISEM = (isemA, isemB)
    GSEM = (gsemA, gsemB)
    SSEM = (ssemA, ssemB)

    def propagate(e_in, e_out):
        @plsc.parallel_loop(0, C, unroll=4)
        def _(i):
            z = jnp.zeros((H,), jnp.float32)
            rowsA[i] = z
            rowsB[i] = z

        nz = RPT // C
        for t in range(nz):
            pltpu.sync_copy(ROWS[t % 2],
                            acc.at[pl.ds(sid * RPT + t * C, C)])
        pltpu.sync_copy(rowsA.at[pl.ds(0, RPT - nz * C)],
                        acc.at[pl.ds(sid * RPT + nz * C, RPT - nz * C)])
        plsc.subcore_barrier()

        def fire_idx(c, b):
            pltpu.async_copy(
                ei3.at[pl.ds(sid * ROWB + c * NSUB, NSUB)], IDXB[b], ISEM[b])

        def wait_idx(b):
            pltpu.make_async_copy(
                ei3.at[pl.ds(0, NSUB)], IDXB[b], ISEM[b]).wait()

        def rebase(b):
            @plsc.parallel_loop(0, NSUB)
            def _(j):
                for g in range(SUB // 16):
                    IDXB[b][j, 0, pl.ds(g * 16, 16)] = (
                        IDXB[b][j, 0, pl.ds(g * 16, 16)] + base_node)

        def fire_gathers(b):
            for j in range(NSUB):
                pltpu.async_copy(e_in.at[IDXB[b].at[j, 0]],
                                 ROWS[b].at[pl.ds(j * SUB, SUB)], GSEM[b])

        def weight_and_scatter(b):
            for j in range(NSUB):
                pltpu.make_async_copy(
                    e_in.at[IDXB[b].at[j, 0]],
                    ROWS[b].at[pl.ds(j * SUB, SUB)], GSEM[b]).wait()

                @plsc.parallel_loop(0, SUB // 16)
                def _(gg, j=j):
                    SIDX[b][j, pl.ds(gg * 16, 16)] = (
                        IDXB[b][j, 1, pl.ds(gg * 16, 16)])
                    wv = plsc.bitcast(IDXB[b][j, 2, pl.ds(gg * 16, 16)],
                                      jnp.float32)
                    base = j * SUB + gg * 16
                    for e in range(16):
                        ROWS[b][base + e] = ROWS[b][base + e] * wv[e]

                pltpu.async_copy(ROWS[b].at[pl.ds(j * SUB, SUB)],
                                 acc.at[SIDX[b].at[j]], SSEM[b], add=True)

        def drain_scatters(b):
            pltpu.make_async_copy(
                ROWS[b], acc.at[pl.ds(0, C)], SSEM[b]).wait()

        fire_idx(0, 0)

        @pl.loop(0, NCHUNK // 2)
        def _(t):
            wait_idx(0)

            @pl.when(t >= 1)
            def _():
                drain_scatters(0)

            rebase(0)
            fire_gathers(0)

            @pl.when(t >= 1)
            def _():
                weight_and_scatter(1)

            fire_idx(2 * t + 1, 1)

            wait_idx(1)

            @pl.when(t >= 1)
            def _():
                drain_scatters(1)

            rebase(1)
            fire_gathers(1)

            weight_and_scatter(0)

            @pl.when(t + 1 < NCHUNK // 2)
            def _():
                fire_idx(2 * t + 2, 0)

        weight_and_scatter(1)
        drain_scatters(0)
        drain_scatters(1)

        plsc.subcore_barrier()
        pltpu.sync_copy(acc.at[pl.ds(sid * RPT, RPT)],
                        e_out.at[pl.ds(base_node + sid * RPT, RPT)])
        plsc.subcore_barrier()

    propagate(e0, e1)
    propagate(e1, e2)

    third = jnp.float32(1.0 / 3.0)

    def lookup(src4, offset, emb_out):
        pltpu.sync_copy(src4.at[sid, 0], sidxA.at[pl.ds(0, FN)])
        pltpu.sync_copy(src4.at[sid, 1], sidxB.at[pl.ds(0, FN)])

        for sb in (sidxA, sidxB):
            @plsc.parallel_loop(0, FN)
            def _(j, sb=sb):
                for g in range(SUB // 16):
                    sb[j, pl.ds(g * 16, 16)] = (
                        sb[j, pl.ds(g * 16, 16)] + (base_node + offset))

        for half, sb in enumerate((sidxA, sidxB)):
            @pl.loop(0, FN)
            def _(f, half=half, sb=sb):
                for ti, tbl in enumerate((e0, e1, e2)):
                    pltpu.async_copy(tbl.at[sb.at[f]],
                                     rowsA.at[pl.ds(ti * SUB, SUB)], gsemA)
                for ti, tbl in enumerate((e0, e1, e2)):
                    pltpu.make_async_copy(
                        tbl.at[sb.at[f]],
                        rowsA.at[pl.ds(ti * SUB, SUB)], gsemA).wait()

                @plsc.parallel_loop(0, SUB, unroll=4)
                def _(i):
                    rowsA[3 * SUB + i] = (
                        (rowsA[i] + rowsA[SUB + i] + rowsA[2 * SUB + i])
                        * third)

                pltpu.sync_copy(
                    rowsA.at[pl.ds(3 * SUB, SUB)],
                    emb_out.at[pl.ds(
                        cid * B + sid * BPT + (half * FN + f) * SUB, SUB)])

    lookup(users4, 0, uemb)
    lookup(items4, N_USERS, iemb)


_MESH = plsc.VectorSubcoreMesh(core_axis_name="c", subcore_axis_name="s",
                               num_cores=NC, num_subcores=NS)

_gcn = functools.partial(
    pl.kernel,
    out_type=(
        jax.ShapeDtypeStruct((2 * NP, H), jnp.float32),
        jax.ShapeDtypeStruct((2 * NP, H), jnp.float32),
        jax.ShapeDtypeStruct((2 * B, H), jnp.float32),
        jax.ShapeDtypeStruct((2 * B, H), jnp.float32),
    ),
    mesh=_MESH,
    scratch_types=[
        pltpu.VMEM_SHARED((NP, H), jnp.float32),
        pltpu.VMEM((C, H), jnp.float32),
        pltpu.VMEM((C, H), jnp.float32),
        pltpu.VMEM((NSUB, 3, SUB), jnp.int32),
        pltpu.VMEM((NSUB, 3, SUB), jnp.int32),
        pltpu.VMEM((NSUB, SUB), jnp.int32),
        pltpu.VMEM((NSUB, SUB), jnp.int32),
        pltpu.SemaphoreType.DMA,
        pltpu.SemaphoreType.DMA,
        pltpu.SemaphoreType.DMA,
        pltpu.SemaphoreType.DMA,
        pltpu.SemaphoreType.DMA,
        pltpu.SemaphoreType.DMA,
    ],
    compiler_params=pltpu.CompilerParams(use_tc_tiling_on_sc=False,
                                         needs_layout_passes=False),
)(_gcn_body)


def _scores_body(u_ref, i_ref, o_ref):
    o_ref[...] = jnp.sum(u_ref[...] * i_ref[...], axis=1)


_scores = pl.pallas_call(
    _scores_body,
    out_shape=jax.ShapeDtypeStruct((B,), jnp.float32),
    grid=(8,),
    in_specs=[pl.BlockSpec((B // 8, 32), lambda i: (i, 0))] * 2,
    out_specs=pl.BlockSpec((B // 8,), lambda i: (i,)),
)


def kernel(user_table, item_table, edge_weight, edge_index, users, items):
    all_emb = jnp.concatenate([user_table, item_table], axis=0)
    npad = jnp.zeros((NP - N_NODES, H), jnp.float32)
    e0 = jnp.concatenate(
        [all_emb[:, :H], npad, all_emb[:, H:], npad], axis=0)
    epad = jnp.zeros((EP - E,), jnp.int32)
    ci = edge_index.astype(jnp.int32)
    colp = jnp.concatenate([ci[1], epad]).reshape(-1, 1, SUB)
    dstp = jnp.concatenate([ci[0], epad]).reshape(-1, 1, SUB)
    wbits = jax.lax.bitcast_convert_type(
        jnp.concatenate([edge_weight, jnp.zeros((EP - E,), jnp.float32)]),
        jnp.int32).reshape(-1, 1, SUB)
    ei3 = jnp.concatenate([colp, dstp, wbits], axis=1)
    users4 = users.astype(jnp.int32).reshape(NS, 2, FN, SUB)
    items4 = items.astype(jnp.int32).reshape(NS, 2, FN, SUB)
    _, _, ue, ie = _gcn(e0, ei3, users4, items4)
    users_emb = jnp.concatenate([ue[:B], ue[B:]], axis=1)
    items_emb = jnp.concatenate([ie[:B], ie[B:]], axis=1)
    scores = _scores(users_emb, items_emb)
    return (users_emb, items_emb, scores)


# speedup vs baseline: 1.0577x; 1.0041x over previous
import functools

import jax
import jax.numpy as jnp
from jax import lax
from jax.experimental import pallas as pl
from jax.experimental.pallas import tpu as pltpu
from jax.experimental.pallas import tpu_sc as plsc

N_USERS = 50000
N_NODES = 100000
NP = 100096
E = 1600000
EP = 1638400
H = 16
NC, NS = 2, 16
EPT = EP // NS
C = 640
NCHUNK = EPT // C
SUB = 128
NSUB = C // SUB
FN = 4
B = 16384
BPT = B // NS
RPT = NP // NS
ROWB = EPT // SUB


def _gcn_body(e0, ei3, users4, items4,
              e1, e2, uemb, iemb,
              acc, rowsA, rowsB, idxA, idxB, sidxA, sidxB,
              isemA, isemB, gsemA, gsemB, ssemA, ssemB):
    cid = lax.axis_index("c")
    sid = lax.axis_index("s")
    base_node = cid * NP
    ROWS = (rowsA, rowsB)
    IDXB = (idxA, idxB)
    SIDX = (sidxA, sidxB)
    ISEM = (isemA, isemB)
    GSEM = (gsemA, gsemB)
    SSEM = (ssemA, ssemB)

    def propagate(e_in, e_out):
        @plsc.parallel_loop(0, C, unroll=4)
        def _(i):
            z = jnp.zeros((H,), jnp.float32)
            rowsA[i] = z
            rowsB[i] = z

        nz = RPT // C
        for t in range(nz):
            pltpu.sync_copy(ROWS[t % 2],
                            acc.at[pl.ds(sid * RPT + t * C, C)])
        pltpu.sync_copy(rowsA.at[pl.ds(0, RPT - nz * C)],
                        acc.at[pl.ds(sid * RPT + nz * C, RPT - nz * C)])
        plsc.subcore_barrier()

        def fire_idx(c, b):
            pltpu.async_copy(
                ei3.at[pl.ds(sid * ROWB + c * NSUB, NSUB)], IDXB[b], ISEM[b])

        def wait_idx(b):
            pltpu.make_async_copy(
                ei3.at[pl.ds(0, NSUB)], IDXB[b], ISEM[b]).wait()

        e_in_half = e_in.at[pl.ds(base_node, NP)]

        def fire_gathers(b):
            for j in range(NSUB):
                pltpu.async_copy(e_in_half.at[IDXB[b].at[j, 0]],
                                 ROWS[b].at[pl.ds(j * SUB, SUB)], GSEM[b])

        def weight_and_scatter(b):
            for j in range(NSUB):
                pltpu.make_async_copy(
                    e_in_half.at[IDXB[b].at[j, 0]],
                    ROWS[b].at[pl.ds(j * SUB, SUB)], GSEM[b]).wait()

                @plsc.parallel_loop(0, SUB // 16)
                def _(gg, j=j):
                    SIDX[b][j, pl.ds(gg * 16, 16)] = (
                        IDXB[b][j, 1, pl.ds(gg * 16, 16)])
                    wv = plsc.bitcast(IDXB[b][j, 2, pl.ds(gg * 16, 16)],
                                      jnp.float32)
                    base = j * SUB + gg * 16
                    for e in range(16):
                        ROWS[b][base + e] = ROWS[b][base + e] * wv[e]

                pltpu.async_copy(ROWS[b].at[pl.ds(j * SUB, SUB)],
                                 acc.at[SIDX[b].at[j]], SSEM[b], add=True)

        def drain_scatters(b):
            pltpu.make_async_copy(
                ROWS[b], acc.at[pl.ds(0, C)], SSEM[b]).wait()

        fire_idx(0, 0)

        @pl.loop(0, NCHUNK // 2)
        def _(t):
            wait_idx(0)

            @pl.when(t >= 1)
            def _():
                drain_scatters(0)

            fire_gathers(0)

            @pl.when(t >= 1)
            def _():
                weight_and_scatter(1)

            fire_idx(2 * t + 1, 1)

            wait_idx(1)

            @pl.when(t >= 1)
            def _():
                drain_scatters(1)

            fire_gathers(1)

            weight_and_scatter(0)

            @pl.when(t + 1 < NCHUNK // 2)
            def _():
                fire_idx(2 * t + 2, 0)

        weight_and_scatter(1)
        drain_scatters(0)
        drain_scatters(1)

        plsc.subcore_barrier()
        pltpu.sync_copy(acc.at[pl.ds(sid * RPT, RPT)],
                        e_out.at[pl.ds(base_node + sid * RPT, RPT)])
        plsc.subcore_barrier()

    propagate(e0, e1)
    propagate(e1, e2)

    third = jnp.float32(1.0 / 3.0)

    def lookup(src4, offset, emb_out):
        pltpu.sync_copy(src4.at[sid, 0], sidxA.at[pl.ds(0, FN)])
        pltpu.sync_copy(src4.at[sid, 1], sidxB.at[pl.ds(0, FN)])

        for sb in (sidxA, sidxB):
            @plsc.parallel_loop(0, FN)
            def _(j, sb=sb):
                for g in range(SUB // 16):
                    sb[j, pl.ds(g * 16, 16)] = (
                        sb[j, pl.ds(g * 16, 16)] + (base_node + offset))

        for half, sb in enumerate((sidxA, sidxB)):
            @pl.loop(0, FN)
            def _(f, half=half, sb=sb):
                for ti, tbl in enumerate((e0, e1, e2)):
                    pltpu.async_copy(tbl.at[sb.at[f]],
                                     rowsA.at[pl.ds(ti * SUB, SUB)], gsemA)
                for ti, tbl in enumerate((e0, e1, e2)):
                    pltpu.make_async_copy(
                        tbl.at[sb.at[f]],
                        rowsA.at[pl.ds(ti * SUB, SUB)], gsemA).wait()

                @plsc.parallel_loop(0, SUB, unroll=4)
                def _(i):
                    rowsA[3 * SUB + i] = (
                        (rowsA[i] + rowsA[SUB + i] + rowsA[2 * SUB + i])
                        * third)

                pltpu.sync_copy(
                    rowsA.at[pl.ds(3 * SUB, SUB)],
                    emb_out.at[pl.ds(
                        cid * B + sid * BPT + (half * FN + f) * SUB, SUB)])

    lookup(users4, 0, uemb)
    lookup(items4, N_USERS, iemb)


_MESH = plsc.VectorSubcoreMesh(core_axis_name="c", subcore_axis_name="s",
                               num_cores=NC, num_subcores=NS)

_gcn = functools.partial(
    pl.kernel,
    out_type=(
        jax.ShapeDtypeStruct((2 * NP, H), jnp.float32),
        jax.ShapeDtypeStruct((2 * NP, H), jnp.float32),
        jax.ShapeDtypeStruct((2 * B, H), jnp.float32),
        jax.ShapeDtypeStruct((2 * B, H), jnp.float32),
    ),
    mesh=_MESH,
    scratch_types=[
        pltpu.VMEM_SHARED((NP, H), jnp.float32),
        pltpu.VMEM((C, H), jnp.float32),
        pltpu.VMEM((C, H), jnp.float32),
        pltpu.VMEM((NSUB, 3, SUB), jnp.int32),
        pltpu.VMEM((NSUB, 3, SUB), jnp.int32),
        pltpu.VMEM((NSUB, SUB), jnp.int32),
        pltpu.VMEM((NSUB, SUB), jnp.int32),
        pltpu.SemaphoreType.DMA,
        pltpu.SemaphoreType.DMA,
        pltpu.SemaphoreType.DMA,
        pltpu.SemaphoreType.DMA,
        pltpu.SemaphoreType.DMA,
        pltpu.SemaphoreType.DMA,
    ],
    compiler_params=pltpu.CompilerParams(use_tc_tiling_on_sc=False,
                                         needs_layout_passes=False),
)(_gcn_body)


def _scores_body(u_ref, i_ref, o_ref):
    o_ref[...] = jnp.sum(u_ref[...] * i_ref[...], axis=1)


_scores = pl.pallas_call(
    _scores_body,
    out_shape=jax.ShapeDtypeStruct((B,), jnp.float32),
    grid=(8,),
    in_specs=[pl.BlockSpec((B // 8, 32), lambda i: (i, 0))] * 2,
    out_specs=pl.BlockSpec((B // 8,), lambda i: (i,)),
)


def kernel(user_table, item_table, edge_weight, edge_index, users, items):
    all_emb = jnp.concatenate([user_table, item_table], axis=0)
    npad = jnp.zeros((NP - N_NODES, H), jnp.float32)
    e0 = jnp.concatenate(
        [all_emb[:, :H], npad, all_emb[:, H:], npad], axis=0)
    epad = jnp.zeros((EP - E,), jnp.int32)
    ci = edge_index.astype(jnp.int32)
    colp = jnp.concatenate([ci[1], epad]).reshape(-1, 1, SUB)
    dstp = jnp.concatenate([ci[0], epad]).reshape(-1, 1, SUB)
    wbits = jax.lax.bitcast_convert_type(
        jnp.concatenate([edge_weight, jnp.zeros((EP - E,), jnp.float32)]),
        jnp.int32).reshape(-1, 1, SUB)
    ei3 = jnp.concatenate([colp, dstp, wbits], axis=1)
    users4 = users.astype(jnp.int32).reshape(NS, 2, FN, SUB)
    items4 = items.astype(jnp.int32).reshape(NS, 2, FN, SUB)
    _, _, ue, ie = _gcn(e0, ei3, users4, items4)
    users_emb = jnp.concatenate([ue[:B], ue[B:]], axis=1)
    items_emb = jnp.concatenate([ie[:B], ie[B:]], axis=1)
    scores = _scores(users_emb, items_emb)
    return (users_emb, items_emb, scores)
